# Initial kernel scaffold; baseline (speedup 1.0000x reference)
import jax
import jax.numpy as jnp
from jax.experimental import pallas as pl


def kernel(words_tensor, pos_tensor, word_table, pos_table, W, b):
    raise NotImplementedError("write your pallas kernel here")


# speedup vs baseline: 4.6024x; 4.6024x over previous
import functools
import jax
import jax.numpy as jnp
from jax import lax
from jax.experimental import pallas as pl
from jax.experimental.pallas import tpu as pltpu
from jax.experimental.pallas import tpu_sc as plsc

WORD_DIM = 128
POS_DIM = 64
OUT_DIM = 128


def _word_fold_body(wt_ref, w_ref, out_ref):
    out_ref[...] = jnp.dot(wt_ref[...], w_ref[...],
                           preferred_element_type=jnp.float32)


def _pos_fold_body(pt_ref, w_ref, b_ref, out_ref):
    out_ref[...] = jnp.dot(pt_ref[...], w_ref[...],
                           preferred_element_type=jnp.float32) + b_ref[...]


def _fold_tables(word_table, pos_table, W, b):
    V_w = word_table.shape[0]
    V_p = pos_table.shape[0]
    ww_t = W[:, :WORD_DIM].T
    wp_t = W[:, WORD_DIM:].T
    BLK = 2000
    word_t = pl.pallas_call(
        _word_fold_body,
        grid=(V_w // BLK,),
        in_specs=[
            pl.BlockSpec((BLK, WORD_DIM), lambda i: (i, 0)),
            pl.BlockSpec((WORD_DIM, OUT_DIM), lambda i: (0, 0)),
        ],
        out_specs=pl.BlockSpec((BLK, OUT_DIM), lambda i: (i, 0)),
        out_shape=jax.ShapeDtypeStruct((V_w, OUT_DIM), jnp.float32),
    )(word_table, ww_t)
    pos_t = pl.pallas_call(
        _pos_fold_body,
        out_shape=jax.ShapeDtypeStruct((V_p, OUT_DIM), jnp.float32),
    )(pos_table, wp_t, b.reshape(1, OUT_DIM))
    return word_t, pos_t


_CHUNK = 128


def _make_sc_gather(n_tokens):
    info = plsc.get_sparse_core_info()
    nw = info.num_cores * info.num_subcores
    per_w = n_tokens // nw
    n_chunks = per_w // _CHUNK
    mesh = plsc.VectorSubcoreMesh(core_axis_name="c", subcore_axis_name="s")

    @functools.partial(
        pl.kernel,
        mesh=mesh,
        out_type=jax.ShapeDtypeStruct((n_tokens, OUT_DIM), jnp.float32),
        scratch_types=[
            pltpu.VMEM((_CHUNK,), jnp.int32),
            pltpu.VMEM((_CHUNK,), jnp.int32),
            pltpu.VMEM((_CHUNK, OUT_DIM), jnp.float32),
            pltpu.VMEM((_CHUNK, OUT_DIM), jnp.float32),
            pltpu.SemaphoreType.DMA,
            pltpu.SemaphoreType.DMA,
        ],
    )
    def sc_kernel(wt_hbm, pt_hbm, widx_hbm, pidx_hbm, out_hbm,
                  widx_v, pidx_v, wrows_v, prows_v, sem_w, sem_p):
        wid = lax.axis_index("s") * info.num_cores + lax.axis_index("c")
        base = wid * per_w

        def chunk_body(c, carry):
            off = base + c * _CHUNK
            pltpu.sync_copy(widx_hbm.at[pl.ds(off, _CHUNK)], widx_v)
            pltpu.sync_copy(pidx_hbm.at[pl.ds(off, _CHUNK)], pidx_v)
            cp_w = pltpu.async_copy(wt_hbm.at[widx_v], wrows_v, sem_w)
            cp_p = pltpu.async_copy(pt_hbm.at[pidx_v], prows_v, sem_p)
            cp_w.wait()
            cp_p.wait()

            def tok_body(t, carry2):
                for j in range(OUT_DIM // 16):
                    s = pl.ds(j * 16, 16)
                    x = wrows_v[t, s] + prows_v[t, s]
                    e = jnp.exp(x + x)
                    wrows_v[t, s] = 1.0 - 2.0 / (e + 1.0)
                return carry2

            lax.fori_loop(0, _CHUNK, tok_body, 0)
            pltpu.sync_copy(wrows_v, out_hbm.at[pl.ds(off, _CHUNK)])
            return carry

        lax.fori_loop(0, n_chunks, chunk_body, 0)

    return sc_kernel


def kernel(words_tensor, pos_tensor, word_table, pos_table, W, b):
    B, L = words_tensor.shape
    n_tokens = B * L
    word_t, pos_t = _fold_tables(word_table, pos_table, W, b)
    widx = words_tensor.reshape(n_tokens).astype(jnp.int32)
    pidx = pos_tensor.reshape(n_tokens).astype(jnp.int32)
    out = _make_sc_gather(n_tokens)(word_t, pos_t, widx, pidx)
    return out.reshape(B, L, OUT_DIM)


# speedup vs baseline: 7.9347x; 1.7241x over previous
import functools
import jax
import jax.numpy as jnp
from jax import lax
from jax.experimental import pallas as pl
from jax.experimental.pallas import tpu as pltpu
from jax.experimental.pallas import tpu_sc as plsc

WORD_DIM = 128
POS_DIM = 64
OUT_DIM = 128


def _word_fold_body(wt_ref, w_ref, out_ref):
    out_ref[...] = jnp.dot(wt_ref[...], w_ref[...],
                           preferred_element_type=jnp.float32)


def _pos_fold_body(pt_ref, w_ref, b_ref, out_ref):
    out_ref[...] = jnp.dot(pt_ref[...], w_ref[...],
                           preferred_element_type=jnp.float32) + b_ref[...]


def _fold_tables(word_table, pos_table, W, b):
    V_w = word_table.shape[0]
    V_p = pos_table.shape[0]
    ww_t = W[:, :WORD_DIM].T
    wp_t = W[:, WORD_DIM:].T
    BLK = 2000
    word_t = pl.pallas_call(
        _word_fold_body,
        grid=(V_w // BLK,),
        in_specs=[
            pl.BlockSpec((BLK, WORD_DIM), lambda i: (i, 0)),
            pl.BlockSpec((WORD_DIM, OUT_DIM), lambda i: (0, 0)),
        ],
        out_specs=pl.BlockSpec((BLK, OUT_DIM), lambda i: (i, 0)),
        out_shape=jax.ShapeDtypeStruct((V_w, OUT_DIM), jnp.float32),
    )(word_table, ww_t)
    pos_t = pl.pallas_call(
        _pos_fold_body,
        out_shape=jax.ShapeDtypeStruct((V_p, OUT_DIM), jnp.float32),
    )(pos_table, wp_t, b.reshape(1, OUT_DIM))
    return word_t, pos_t


_CHUNK = 128


def _make_sc_gather(n_tokens):
    info = plsc.get_sparse_core_info()
    nw = info.num_cores * info.num_subcores
    per_w = n_tokens // nw
    n_chunks = per_w // _CHUNK
    mesh = plsc.VectorSubcoreMesh(core_axis_name="c", subcore_axis_name="s")

    @functools.partial(
        pl.kernel,
        mesh=mesh,
        out_type=jax.ShapeDtypeStruct((n_tokens, OUT_DIM), jnp.float32),
        scratch_types=[
            pltpu.VMEM((n_chunks, _CHUNK), jnp.int32),
            pltpu.VMEM((n_chunks, _CHUNK), jnp.int32),
            pltpu.VMEM((2, _CHUNK, OUT_DIM), jnp.float32),
            pltpu.VMEM((2, _CHUNK, OUT_DIM), jnp.float32),
            pltpu.SemaphoreType.DMA,
            pltpu.SemaphoreType.DMA,
        ],
    )
    def sc_kernel(wt_hbm, pt_hbm, widx_hbm, pidx_hbm, out_hbm,
                  widx_v, pidx_v, wrows_v, prows_v, sem_w, sem_p):
        wid = lax.axis_index("s") * info.num_cores + lax.axis_index("c")
        base = wid * per_w
        row_base = wid * n_chunks

        pltpu.sync_copy(widx_hbm.at[pl.ds(row_base, n_chunks)], widx_v)
        pltpu.sync_copy(pidx_hbm.at[pl.ds(row_base, n_chunks)], pidx_v)

        def issue(c, buf):
            cp_w = pltpu.async_copy(wt_hbm.at[widx_v.at[c]],
                                    wrows_v.at[buf], sem_w)
            cp_p = pltpu.async_copy(pt_hbm.at[pidx_v.at[c]],
                                    prows_v.at[buf], sem_p)
            return cp_w, cp_p

        def drain(buf):
            pltpu.make_async_copy(wt_hbm.at[widx_v.at[0]],
                                  wrows_v.at[buf], sem_w).wait()
            pltpu.make_async_copy(pt_hbm.at[pidx_v.at[0]],
                                  prows_v.at[buf], sem_p).wait()

        def compute_store(c, buf):
            wb = wrows_v.at[buf]
            pb = prows_v.at[buf]

            def tok_body(t, carry):
                for j in range(OUT_DIM // 16):
                    s = pl.ds(j * 16, 16)
                    x = wb[t, s] + pb[t, s]
                    e = jnp.exp(x + x)
                    wb[t, s] = 1.0 - 2.0 / (e + 1.0)
                return carry

            lax.fori_loop(0, _CHUNK, tok_body, 0)
            pltpu.sync_copy(wb, out_hbm.at[pl.ds(base + c * _CHUNK, _CHUNK)])

        issue(0, 0)

        def outer(c0, carry):
            for b in range(2):
                c = c0 * 2 + b

                @pl.when(c + 1 < n_chunks)
                def _():
                    issue(c + 1, (b + 1) % 2)

                drain(b)
                compute_store(c, b)
            return carry

        lax.fori_loop(0, n_chunks // 2, outer, 0)

    return sc_kernel


def kernel(words_tensor, pos_tensor, word_table, pos_table, W, b):
    B, L = words_tensor.shape
    n_tokens = B * L
    word_t, pos_t = _fold_tables(word_table, pos_table, W, b)
    widx = words_tensor.reshape(n_tokens // _CHUNK, _CHUNK).astype(jnp.int32)
    pidx = pos_tensor.reshape(n_tokens // _CHUNK, _CHUNK).astype(jnp.int32)
    out = _make_sc_gather(n_tokens)(word_t, pos_t, widx, pidx)
    return out.reshape(B, L, OUT_DIM)
